# baseline (speedup 1.0000x reference)
import functools
import numpy as np
import jax
import jax.numpy as jnp
from jax import lax
from jax.experimental import pallas as pl
from jax.experimental.pallas import tpu as pltpu
from jax.experimental.pallas import tpu_sc as plsc

_P = 0.5
_MAX_DROP = 8


def _drop_indices():
    rng = np.random.RandomState(1)
    if not (rng.rand() < _P):
        return np.zeros((0,), np.int32)
    num_drop = int(rng.randint(1, _MAX_DROP + 1))
    return np.sort(rng.permutation(96)[:num_drop].astype(np.int32))


_DROP = tuple(int(i) for i in _drop_indices())

_B, _C, _H, _W = 16, 96, 224, 224
_NC, _NS = 2, 16
_NW = _NC * _NS
_PPW = (_B * _C) // _NW
_HH = _H // 2
_NBUF = 4
_NQ = _PPW * 2
_ZROWS = 56


def _sc_body(in_hbm, out_hbm, buf, zbuf, gsems, ssems, zsem):
    wid = lax.axis_index("s") * _NC + lax.axis_index("c")
    bb = wid // 2
    c_base = (wid % 2) * _PPW

    zv = jnp.zeros((16,), jnp.float32)
    for i in range(_ZROWS):
        for j in range(_W // 16):
            zbuf[i, pl.ds(j * 16, 16)] = zv

    def gather(q, j):
        return pltpu.make_async_copy(
            in_hbm.at[bb, c_base + q // 2, pl.ds((q % 2) * _HH, _HH)],
            buf.at[j], gsems.at[j])

    def scatter(q, j):
        return pltpu.make_async_copy(
            buf.at[j],
            out_hbm.at[bb, c_base + q // 2, pl.ds((q % 2) * _HH, _HH)],
            ssems.at[j])

    for j in range(_NBUF):
        gather(j, j).start()

    def group(q0, start_next):
        for j in range(_NBUF):
            q = q0 + j
            gather(q, j).wait()
            scatter(q, j).start()
        for j in range(_NBUF):
            q = q0 + j
            scatter(q, j).wait()
            if start_next:
                gather(q + _NBUF, j).start()

    def step(k, _):
        group(_NBUF * k, True)
        return 0
    lax.fori_loop(0, _NQ // _NBUF - 1, step, 0)
    group(_NQ - _NBUF, False)

    for b in range(_B):
        for d in _DROP:
            owner = 2 * b + (1 if d >= _PPW else 0)

            @pl.when(wid == owner)
            def _():
                cps = []
                for r0 in range(0, _H, _ZROWS):
                    cp = pltpu.make_async_copy(
                        zbuf, out_hbm.at[b, d, pl.ds(r0, _ZROWS)], zsem)
                    cp.start()
                    cps.append(cp)
                for cp in cps:
                    cp.wait()


_mesh = plsc.VectorSubcoreMesh(core_axis_name="c", subcore_axis_name="s")

_sc_kernel = functools.partial(
    pl.kernel,
    out_type=jax.ShapeDtypeStruct((_B, _C, _H, _W), jnp.float32),
    mesh=_mesh,
    scratch_types=[
        pltpu.VMEM((_NBUF, _HH, _W), jnp.float32),
        pltpu.VMEM((_ZROWS, _W), jnp.float32),
        pltpu.SemaphoreType.DMA((_NBUF,)),
        pltpu.SemaphoreType.DMA((_NBUF,)),
        pltpu.SemaphoreType.DMA,
    ],
)(_sc_body)


def kernel(image):
    return _sc_kernel(image)


# speedup vs baseline: 1.3075x; 1.3075x over previous
import numpy as np
import jax
import jax.numpy as jnp
from jax.experimental import pallas as pl
from jax.experimental.pallas import tpu as pltpu

_P = 0.5
_MAX_DROP = 8


def _drop_indices():
    rng = np.random.RandomState(1)
    if not (rng.rand() < _P):
        return np.zeros((0,), np.int32)
    num_drop = int(rng.randint(1, _MAX_DROP + 1))
    return np.sort(rng.permutation(96)[:num_drop].astype(np.int32))


_DROP = tuple(int(i) for i in _drop_indices())

_B, _C, _H, _W = 16, 96, 224, 224
_HC = _C // 2
_NQ = 2 * _B
_NBUF = 4


def _runs_in(lo, hi):
    runs, prev = [], lo
    for d in _DROP:
        if lo <= d < hi:
            if d > prev:
                runs.append((prev, d - prev))
            prev = d + 1
    if prev < hi:
        runs.append((prev, hi - prev))
    return runs


_HALF_RUNS = (_runs_in(0, _HC), _runs_in(_HC, _C))
_HALF_DROPS = (
    tuple(d for d in _DROP if d < _HC),
    tuple(d for d in _DROP if d >= _HC),
)


def _body(in_hbm, out_hbm, buf, in_sems, out_sems):
    for j in range(_NBUF):
        for d in _HALF_DROPS[j % 2]:
            buf[j, d % _HC] = jnp.zeros((_H, _W), jnp.float32)

    def start_in(q):
        j = q % _NBUF
        b, h = q // 2, q % 2
        cps = []
        for c0, ln in _HALF_RUNS[h]:
            cp = pltpu.make_async_copy(
                in_hbm.at[b, pl.ds(c0, ln)],
                buf.at[j, pl.ds(c0 - h * _HC, ln)],
                in_sems.at[j],
            )
            cp.start()
            cps.append(cp)
        return cps

    def start_out(q):
        j = q % _NBUF
        b, h = q // 2, q % 2
        cp = pltpu.make_async_copy(
            buf.at[j], out_hbm.at[b, pl.ds(h * _HC, _HC)], out_sems.at[j])
        cp.start()
        return cp

    copies_in = {0: start_in(0)}
    copies_out = {}
    for q in range(_NQ):
        if q + 1 < _NQ:
            if q >= _NBUF - 1:
                copies_out[q - (_NBUF - 1)].wait()
            copies_in[q + 1] = start_in(q + 1)
        for cp in copies_in[q]:
            cp.wait()
        copies_out[q] = start_out(q)
    for q in range(_NQ - _NBUF, _NQ):
        copies_out[q].wait()


def kernel(image):
    return pl.pallas_call(
        _body,
        in_specs=[pl.BlockSpec(memory_space=pl.ANY)],
        out_specs=pl.BlockSpec(memory_space=pl.ANY),
        out_shape=jax.ShapeDtypeStruct((_B, _C, _H, _W), jnp.float32),
        scratch_shapes=[
            pltpu.VMEM((_NBUF, _HC, _H, _W), jnp.float32),
            pltpu.SemaphoreType.DMA((_NBUF,)),
            pltpu.SemaphoreType.DMA((_NBUF,)),
        ],
    )(image)
